# Initial kernel scaffold; baseline (speedup 1.0000x reference)
import jax
import jax.numpy as jnp
from jax.experimental import pallas as pl


def kernel(local_hidden_states, up_weight, full_topk_ids):
    raise NotImplementedError("write your pallas kernel here")


# speedup vs baseline: 1.1972x; 1.1972x over previous
import jax
import jax.numpy as jnp
from jax.experimental import pallas as pl
from jax.experimental.pallas import tpu as pltpu

_BM = 128


def _gemm_silu_kernel(block_expert_ref, x_ref, wg_ref, wu_ref, o_ref):
    a = x_ref[...]
    g = jax.lax.dot_general(
        a, wg_ref[0], (((1,), (1,)), ((), ())), preferred_element_type=jnp.float32
    )
    u = jax.lax.dot_general(
        a, wu_ref[0], (((1,), (1,)), ((), ())), preferred_element_type=jnp.float32
    )
    o_ref[...] = g * jax.nn.sigmoid(g) * u


def kernel(local_hidden_states, up_weight, full_topk_ids):
    T, K = local_hidden_states.shape
    E, N, _ = up_weight.shape
    topk = full_topk_ids.shape[1]
    M = T * topk
    N2 = N // 2

    flat_ids = full_topk_ids.reshape(-1).astype(jnp.int32)

    sort_idx = jnp.argsort(flat_ids)
    sorted_ids = flat_ids[sort_idx]
    counts = jnp.bincount(flat_ids, length=E)
    padded_counts = ((counts + _BM - 1) // _BM) * _BM
    padded_starts = jnp.concatenate(
        [jnp.zeros((1,), jnp.int32), jnp.cumsum(padded_counts)[:-1].astype(jnp.int32)]
    )
    unpadded_starts = jnp.concatenate(
        [jnp.zeros((1,), jnp.int32), jnp.cumsum(counts)[:-1].astype(jnp.int32)]
    )
    dest = padded_starts[sorted_ids] + (
        jnp.arange(M, dtype=jnp.int32) - unpadded_starts[sorted_ids]
    )

    M_pad = M + E * _BM
    num_blocks = M_pad // _BM

    row_map = jnp.zeros((M_pad,), jnp.int32).at[dest].set(sort_idx // topk)
    inv_map = jnp.zeros((M,), jnp.int32).at[sort_idx].set(dest)
    block_expert = jnp.searchsorted(
        jnp.cumsum(padded_counts),
        jnp.arange(num_blocks, dtype=jnp.int32) * _BM,
        side="right",
    ).astype(jnp.int32)
    block_expert = jnp.minimum(block_expert, E - 1)

    x_sorted = local_hidden_states[row_map]

    wg = up_weight[:, :N2, :]
    wu = up_weight[:, N2:, :]

    grid_spec = pltpu.PrefetchScalarGridSpec(
        num_scalar_prefetch=1,
        grid=(num_blocks,),
        in_specs=[
            pl.BlockSpec((_BM, K), lambda i, be: (i, 0)),
            pl.BlockSpec((1, N2, K), lambda i, be: (be[i], 0, 0)),
            pl.BlockSpec((1, N2, K), lambda i, be: (be[i], 0, 0)),
        ],
        out_specs=pl.BlockSpec((_BM, N2), lambda i, be: (i, 0)),
    )
    out_sorted = pl.pallas_call(
        _gemm_silu_kernel,
        grid_spec=grid_spec,
        out_shape=jax.ShapeDtypeStruct((M_pad, N2), jnp.float32),
        compiler_params=pltpu.CompilerParams(
            dimension_semantics=("arbitrary",),
        ),
    )(block_expert, x_sorted, wg, wu)

    return out_sorted[inv_map]


# speedup vs baseline: 1.5071x; 1.2589x over previous
import jax
import jax.numpy as jnp
from jax.experimental import pallas as pl
from jax.experimental.pallas import tpu as pltpu

_BM = 128


def _gemm_silu_kernel(block_expert_ref, x_ref, wg_ref, wu_ref, o_ref):
    a = x_ref[...]
    g = jax.lax.dot_general(
        a, wg_ref[0], (((1,), (1,)), ((), ())), preferred_element_type=jnp.float32
    )
    u = jax.lax.dot_general(
        a, wu_ref[0], (((1,), (1,)), ((), ())), preferred_element_type=jnp.float32
    )
    o_ref[...] = g * jax.nn.sigmoid(g) * u


def kernel(local_hidden_states, up_weight, full_topk_ids):
    T, K = local_hidden_states.shape
    E, N, _ = up_weight.shape
    topk = full_topk_ids.shape[1]
    M = T * topk
    N2 = N // 2

    flat_ids = full_topk_ids.reshape(-1).astype(jnp.int32)

    occ = (flat_ids[:, None] == jnp.arange(E, dtype=jnp.int32)[None, :]).astype(
        jnp.int32
    )
    csum = jnp.cumsum(occ, axis=0)
    counts = csum[-1]
    rank = jnp.sum(csum * occ, axis=1) - 1
    padded_counts = ((counts + _BM - 1) // _BM) * _BM
    cum_padded = jnp.cumsum(padded_counts)
    padded_starts = (cum_padded - padded_counts).astype(jnp.int32)
    dest = jnp.sum(occ * padded_starts[None, :], axis=1) + rank

    M_pad = M + E * _BM
    num_blocks = M_pad // _BM

    row_map = (
        jnp.zeros((M_pad,), jnp.int32)
        .at[dest]
        .set((jnp.arange(M, dtype=jnp.int32) // topk))
    )
    block_expert = jnp.minimum(
        jnp.sum(
            (jnp.arange(num_blocks, dtype=jnp.int32)[:, None] * _BM)
            >= cum_padded[None, :],
            axis=1,
        ).astype(jnp.int32),
        E - 1,
    )

    x_sorted = local_hidden_states[row_map]

    wg = up_weight[:, :N2, :]
    wu = up_weight[:, N2:, :]

    grid_spec = pltpu.PrefetchScalarGridSpec(
        num_scalar_prefetch=1,
        grid=(num_blocks,),
        in_specs=[
            pl.BlockSpec((_BM, K), lambda i, be: (i, 0)),
            pl.BlockSpec((1, N2, K), lambda i, be: (be[i], 0, 0)),
            pl.BlockSpec((1, N2, K), lambda i, be: (be[i], 0, 0)),
        ],
        out_specs=pl.BlockSpec((_BM, N2), lambda i, be: (i, 0)),
    )
    out_sorted = pl.pallas_call(
        _gemm_silu_kernel,
        grid_spec=grid_spec,
        out_shape=jax.ShapeDtypeStruct((M_pad, N2), jnp.float32),
        compiler_params=pltpu.CompilerParams(
            dimension_semantics=("arbitrary",),
        ),
    )(block_expert, x_sorted, wg, wu)

    return out_sorted[dest]


# speedup vs baseline: 1.5321x; 1.0165x over previous
import functools

import jax
import jax.numpy as jnp
from jax.experimental import pallas as pl
from jax.experimental.pallas import tpu as pltpu

_BM = 128
_SUB = 32
_LANE = 128


def _masked_shift(v, s, axis):
    rolled = jnp.roll(v, s, axis=axis)
    idx = jax.lax.broadcasted_iota(jnp.int32, v.shape, axis)
    return jnp.where(idx >= s, rolled, 0)


def _cumsum2d(m):
    s = 1
    while s < _LANE:
        m = m + _masked_shift(m, s, 1)
        s *= 2
    row_tot = jax.lax.broadcast_in_dim(m[:, _LANE - 1], (_SUB, 1), (0,))
    row_tot = jnp.broadcast_to(row_tot, (_SUB, _LANE))
    carry = _masked_shift(row_tot, 1, 0)
    s = 1
    while s < _SUB:
        carry = carry + _masked_shift(carry, s, 0)
        s *= 2
    return m + carry


def _routing_kernel(ids_ref, dest_ref, bexp_ref, E: int):
    ids = ids_ref[...]
    dest = jnp.zeros((_SUB, _LANE), jnp.int32)
    bexp = jnp.zeros((1, _LANE), jnp.int32)
    blk_iota = jax.lax.broadcasted_iota(jnp.int32, (1, _LANE), 1) * _BM
    padded_start = jnp.int32(0)
    for e in range(E):
        m = (ids == e).astype(jnp.int32)
        csum = _cumsum2d(m)
        count = csum[_SUB - 1, _LANE - 1]
        dest = dest + m * (padded_start + csum - 1)
        padded_start = padded_start + ((count + _BM - 1) // _BM) * _BM
        if e < E - 1:
            bexp = bexp + (blk_iota >= padded_start).astype(jnp.int32)
    dest_ref[...] = dest
    bexp_ref[...] = jnp.minimum(bexp, E - 1)


def _gemm_silu_kernel(block_expert_ref, x_ref, wg_ref, wu_ref, o_ref):
    a = x_ref[...]
    g = jax.lax.dot_general(
        a, wg_ref[0], (((1,), (1,)), ((), ())), preferred_element_type=jnp.float32
    )
    u = jax.lax.dot_general(
        a, wu_ref[0], (((1,), (1,)), ((), ())), preferred_element_type=jnp.float32
    )
    o_ref[...] = g * jax.nn.sigmoid(g) * u


def kernel(local_hidden_states, up_weight, full_topk_ids):
    T, K = local_hidden_states.shape
    E, N, _ = up_weight.shape
    topk = full_topk_ids.shape[1]
    M = T * topk
    N2 = N // 2

    ids2d = full_topk_ids.reshape(_SUB, _LANE).astype(jnp.int32)

    M_pad = M + E * _BM
    num_blocks = M_pad // _BM

    dest2d, bexp = pl.pallas_call(
        functools.partial(_routing_kernel, E=E),
        out_shape=(
            jax.ShapeDtypeStruct((_SUB, _LANE), jnp.int32),
            jax.ShapeDtypeStruct((1, _LANE), jnp.int32),
        ),
    )(ids2d)
    dest = dest2d.reshape(M)
    block_expert = bexp[0, :num_blocks]

    dest_cols = dest.reshape(T, topk)
    x_sorted = jnp.zeros((M_pad, K), jnp.float32)
    for j in range(topk):
        x_sorted = x_sorted.at[dest_cols[:, j]].set(local_hidden_states)

    wg = up_weight[:, :N2, :]
    wu = up_weight[:, N2:, :]

    grid_spec = pltpu.PrefetchScalarGridSpec(
        num_scalar_prefetch=1,
        grid=(num_blocks,),
        in_specs=[
            pl.BlockSpec((_BM, K), lambda i, be: (i, 0)),
            pl.BlockSpec((1, N2, K), lambda i, be: (be[i], 0, 0)),
            pl.BlockSpec((1, N2, K), lambda i, be: (be[i], 0, 0)),
        ],
        out_specs=pl.BlockSpec((_BM, N2), lambda i, be: (i, 0)),
    )
    out_sorted = pl.pallas_call(
        _gemm_silu_kernel,
        grid_spec=grid_spec,
        out_shape=jax.ShapeDtypeStruct((M_pad, N2), jnp.float32),
        compiler_params=pltpu.CompilerParams(
            dimension_semantics=("arbitrary",),
        ),
    )(block_expert, x_sorted, wg, wu)

    return out_sorted[dest]


# speedup vs baseline: 1.8356x; 1.1981x over previous
import functools

import jax
import jax.numpy as jnp
from jax import lax
from jax.experimental import pallas as pl
from jax.experimental.pallas import tpu as pltpu
from jax.experimental.pallas import tpu_sc as plsc

_BM = 128
_SUB = 32
_LANE = 128


def _masked_shift(v, s, axis):
    rolled = jnp.roll(v, s, axis=axis)
    idx = jax.lax.broadcasted_iota(jnp.int32, v.shape, axis)
    return jnp.where(idx >= s, rolled, 0)


def _cumsum2d(m):
    s = 1
    while s < _LANE:
        m = m + _masked_shift(m, s, 1)
        s *= 2
    row_tot = jax.lax.broadcast_in_dim(m[:, _LANE - 1], (_SUB, 1), (0,))
    row_tot = jnp.broadcast_to(row_tot, (_SUB, _LANE))
    carry = _masked_shift(row_tot, 1, 0)
    s = 1
    while s < _SUB:
        carry = carry + _masked_shift(carry, s, 0)
        s *= 2
    return m + carry


def _routing_kernel(ids_ref, dest_ref, bexp_ref, E: int):
    ids = ids_ref[...]
    dest = jnp.zeros((_SUB, _LANE), jnp.int32)
    bexp = jnp.zeros((1, _LANE), jnp.int32)
    blk_iota = jax.lax.broadcasted_iota(jnp.int32, (1, _LANE), 1) * _BM
    padded_start = jnp.int32(0)
    for e in range(E):
        m = (ids == e).astype(jnp.int32)
        csum = _cumsum2d(m)
        count = csum[_SUB - 1, _LANE - 1]
        dest = dest + m * (padded_start + csum - 1)
        padded_start = padded_start + ((count + _BM - 1) // _BM) * _BM
        if e < E - 1:
            bexp = bexp + (blk_iota >= padded_start).astype(jnp.int32)
    dest_ref[...] = dest
    bexp_ref[...] = jnp.minimum(bexp, E - 1)


def _gemm_silu_kernel(block_expert_ref, x_ref, wg_ref, wu_ref, o_ref):
    a = x_ref[...]
    g = jax.lax.dot_general(
        a, wg_ref[0], (((1,), (1,)), ((), ())), preferred_element_type=jnp.float32
    )
    u = jax.lax.dot_general(
        a, wu_ref[0], (((1,), (1,)), ((), ())), preferred_element_type=jnp.float32
    )
    o_ref[...] = g * jax.nn.sigmoid(g) * u


def kernel(local_hidden_states, up_weight, full_topk_ids):
    T, K = local_hidden_states.shape
    E, N, _ = up_weight.shape
    topk = full_topk_ids.shape[1]
    M = T * topk
    N2 = N // 2

    ids2d = full_topk_ids.reshape(_SUB, _LANE).astype(jnp.int32)

    M_pad = M + E * _BM
    num_blocks = M_pad // _BM

    dest2d, bexp = pl.pallas_call(
        functools.partial(_routing_kernel, E=E),
        out_shape=(
            jax.ShapeDtypeStruct((_SUB, _LANE), jnp.int32),
            jax.ShapeDtypeStruct((1, _LANE), jnp.int32),
        ),
    )(ids2d)
    dest = dest2d.reshape(M)
    block_expert = bexp[0, :num_blocks]

    dest_t = dest.reshape(T, topk).T
    info = plsc.get_sparse_core_info()
    nw = info.num_cores * info.num_subcores
    t_per_w = T // nw

    @functools.partial(
        pl.kernel,
        mesh=plsc.VectorSubcoreMesh(core_axis_name="c", subcore_axis_name="s"),
        out_type=jax.ShapeDtypeStruct((M_pad, K), jnp.float32),
        scratch_types=[
            pltpu.VMEM((t_per_w,), jnp.int32),
            pltpu.VMEM((t_per_w,), jnp.int32),
            pltpu.VMEM((t_per_w, K), jnp.float32),
            pltpu.SemaphoreType.DMA,
            pltpu.SemaphoreType.DMA,
        ],
    )
    def _dispatch(x_hbm, dest_hbm, out_hbm, idx0, idx1, xv, sem0, sem1):
        wid = lax.axis_index("s") * info.num_cores + lax.axis_index("c")
        base = wid * t_per_w
        pltpu.sync_copy(dest_hbm.at[0, pl.ds(base, t_per_w)], idx0)
        pltpu.sync_copy(dest_hbm.at[1, pl.ds(base, t_per_w)], idx1)
        pltpu.sync_copy(x_hbm.at[pl.ds(base, t_per_w)], xv)
        c0 = pltpu.async_copy(xv, out_hbm.at[idx0], sem0)
        c1 = pltpu.async_copy(xv, out_hbm.at[idx1], sem1)
        c0.wait()
        c1.wait()

    x_sorted = _dispatch(local_hidden_states, dest_t)

    wg = up_weight[:, :N2, :]
    wu = up_weight[:, N2:, :]

    grid_spec = pltpu.PrefetchScalarGridSpec(
        num_scalar_prefetch=1,
        grid=(num_blocks,),
        in_specs=[
            pl.BlockSpec((_BM, K), lambda i, be: (i, 0)),
            pl.BlockSpec((1, N2, K), lambda i, be: (be[i], 0, 0)),
            pl.BlockSpec((1, N2, K), lambda i, be: (be[i], 0, 0)),
        ],
        out_specs=pl.BlockSpec((_BM, N2), lambda i, be: (i, 0)),
    )
    out_sorted = pl.pallas_call(
        _gemm_silu_kernel,
        grid_spec=grid_spec,
        out_shape=jax.ShapeDtypeStruct((M_pad, N2), jnp.float32),
        compiler_params=pltpu.CompilerParams(
            dimension_semantics=("parallel",),
        ),
    )(block_expert, x_sorted, wg, wu)

    return out_sorted[dest]


# speedup vs baseline: 2.3748x; 1.2938x over previous
import functools

import jax
import jax.numpy as jnp
from jax import lax
from jax.experimental import pallas as pl
from jax.experimental.pallas import tpu as pltpu
from jax.experimental.pallas import tpu_sc as plsc

_BM = 128
_SUB = 32
_LANE = 128


def _masked_shift(v, s, axis):
    rolled = jnp.roll(v, s, axis=axis)
    idx = jax.lax.broadcasted_iota(jnp.int32, v.shape, axis)
    return jnp.where(idx >= s, rolled, 0)


def _cumsum2d(m):
    s = 1
    while s < _LANE:
        m = m + _masked_shift(m, s, 1)
        s *= 2
    row_tot = jax.lax.broadcast_in_dim(m[:, _LANE - 1], (_SUB, 1), (0,))
    row_tot = jnp.broadcast_to(row_tot, (_SUB, _LANE))
    carry = _masked_shift(row_tot, 1, 0)
    s = 1
    while s < _SUB:
        carry = carry + _masked_shift(carry, s, 0)
        s *= 2
    return m + carry


def _routing_kernel(ids_ref, dest_ref, bexp_ref, E: int):
    ids = ids_ref[...]
    dest = jnp.zeros((_SUB, _LANE), jnp.int32)
    bexp = jnp.zeros((1, _LANE), jnp.int32)
    blk_iota = jax.lax.broadcasted_iota(jnp.int32, (1, _LANE), 1) * _BM
    padded_start = jnp.int32(0)
    for e in range(E):
        m = (ids == e).astype(jnp.int32)
        csum = _cumsum2d(m)
        count = csum[_SUB - 1, _LANE - 1]
        dest = dest + m * (padded_start + csum - 1)
        padded_start = padded_start + ((count + _BM - 1) // _BM) * _BM
        if e < E - 1:
            bexp = bexp + (blk_iota >= padded_start).astype(jnp.int32)
    dest_ref[...] = dest
    bexp_ref[...] = jnp.minimum(bexp, E - 1)


def _gemm_silu_kernel(block_expert_ref, x_ref, w_ref, o_ref):
    a = x_ref[...]
    g = jax.lax.dot_general(
        a, w_ref[0, 0], (((1,), (1,)), ((), ())), preferred_element_type=jnp.float32
    )
    u = jax.lax.dot_general(
        a, w_ref[0, 1], (((1,), (1,)), ((), ())), preferred_element_type=jnp.float32
    )
    o_ref[...] = g * jax.nn.sigmoid(g) * u


def kernel(local_hidden_states, up_weight, full_topk_ids):
    T, K = local_hidden_states.shape
    E, N, _ = up_weight.shape
    topk = full_topk_ids.shape[1]
    M = T * topk
    N2 = N // 2

    ids2d = full_topk_ids.reshape(_SUB, _LANE).astype(jnp.int32)

    M_pad = M + E * _BM
    num_blocks = M_pad // _BM

    dest2d, bexp = pl.pallas_call(
        functools.partial(_routing_kernel, E=E),
        out_shape=(
            jax.ShapeDtypeStruct((_SUB, _LANE), jnp.int32),
            jax.ShapeDtypeStruct((1, _LANE), jnp.int32),
        ),
    )(ids2d)
    dest = dest2d.reshape(M)
    block_expert = bexp[0, :num_blocks]

    dest_t = dest.reshape(T, topk).T
    info = plsc.get_sparse_core_info()
    nw = info.num_cores * info.num_subcores
    t_per_w = T // nw

    @functools.partial(
        pl.kernel,
        mesh=plsc.VectorSubcoreMesh(core_axis_name="c", subcore_axis_name="s"),
        out_type=jax.ShapeDtypeStruct((M_pad, K), jnp.float32),
        scratch_types=[
            pltpu.VMEM((t_per_w,), jnp.int32),
            pltpu.VMEM((t_per_w,), jnp.int32),
            pltpu.VMEM((t_per_w, K), jnp.float32),
            pltpu.SemaphoreType.DMA,
            pltpu.SemaphoreType.DMA,
        ],
    )
    def _dispatch(x_hbm, dest_hbm, out_hbm, idx0, idx1, xv, sem0, sem1):
        wid = lax.axis_index("s") * info.num_cores + lax.axis_index("c")
        base = wid * t_per_w
        pltpu.sync_copy(dest_hbm.at[0, pl.ds(base, t_per_w)], idx0)
        pltpu.sync_copy(dest_hbm.at[1, pl.ds(base, t_per_w)], idx1)
        pltpu.sync_copy(x_hbm.at[pl.ds(base, t_per_w)], xv)
        c0 = pltpu.async_copy(xv, out_hbm.at[idx0], sem0)
        c1 = pltpu.async_copy(xv, out_hbm.at[idx1], sem1)
        c0.wait()
        c1.wait()

    x_sorted = _dispatch(local_hidden_states, dest_t)

    w4 = up_weight.reshape(E, 2, N2, K)

    grid_spec = pltpu.PrefetchScalarGridSpec(
        num_scalar_prefetch=1,
        grid=(num_blocks,),
        in_specs=[
            pl.BlockSpec((_BM, K), lambda i, be: (i, 0)),
            pl.BlockSpec((1, 2, N2, K), lambda i, be: (be[i], 0, 0, 0)),
        ],
        out_specs=pl.BlockSpec((_BM, N2), lambda i, be: (i, 0)),
    )
    out_sorted = pl.pallas_call(
        _gemm_silu_kernel,
        grid_spec=grid_spec,
        out_shape=jax.ShapeDtypeStruct((M_pad, N2), jnp.float32),
        compiler_params=pltpu.CompilerParams(
            dimension_semantics=("parallel",),
        ),
    )(block_expert, x_sorted, w4)

    return out_sorted[dest]


# speedup vs baseline: 2.5397x; 1.0694x over previous
import functools

import jax
import jax.numpy as jnp
from jax import lax
from jax.experimental import pallas as pl
from jax.experimental.pallas import tpu as pltpu
from jax.experimental.pallas import tpu_sc as plsc

_BM = 128
_SUB = 32
_LANE = 128


def _masked_shift(v, s, axis):
    rolled = jnp.roll(v, s, axis=axis)
    idx = jax.lax.broadcasted_iota(jnp.int32, v.shape, axis)
    return jnp.where(idx >= s, rolled, 0)


def _cumsum2d(m):
    s = 1
    while s < _LANE:
        m = m + _masked_shift(m, s, 1)
        s *= 2
    row_tot = jax.lax.broadcast_in_dim(m[:, _LANE - 1], (_SUB, 1), (0,))
    row_tot = jnp.broadcast_to(row_tot, (_SUB, _LANE))
    carry = _masked_shift(row_tot, 1, 0)
    s = 1
    while s < _SUB:
        carry = carry + _masked_shift(carry, s, 0)
        s *= 2
    return m + carry


_NBUF = 6
_LOOKAHEAD = 5


def _routing_kernel(ids_ref, dest_ref, bexp_ref, first_ref, slot_ref, E: int):
    ids = ids_ref[...]
    dest = jnp.zeros((_SUB, _LANE), jnp.int32)
    bexp = jnp.zeros((1, _LANE), jnp.int32)
    blk_iota = jax.lax.broadcasted_iota(jnp.int32, (1, _LANE), 1) * _BM
    padded_start = jnp.int32(0)
    for e in range(E):
        m = (ids == e).astype(jnp.int32)
        csum = _cumsum2d(m)
        count = csum[_SUB - 1, _LANE - 1]
        dest = dest + m * (padded_start + csum - 1)
        padded_start = padded_start + ((count + _BM - 1) // _BM) * _BM
        if e < E - 1:
            bexp = bexp + (blk_iota >= padded_start).astype(jnp.int32)
    bexp = jnp.minimum(bexp, E - 1)
    lane = jax.lax.broadcasted_iota(jnp.int32, (1, _LANE), 1)
    first = (bexp != _masked_shift(bexp, 1, 1)).astype(jnp.int32)
    first = jnp.where(lane == 0, 1, first)
    runidx = first
    s = 1
    while s < _LANE:
        runidx = runidx + _masked_shift(runidx, s, 1)
        s *= 2
    runidx = runidx - 1
    dest_ref[...] = dest
    bexp_ref[...] = bexp
    first_ref[...] = first
    slot_ref[...] = runidx % _NBUF


def _gemm_silu_kernel(be_ref, first_ref, slot_ref, x_ref, w_hbm, o_ref, wbuf, sems):
    i = pl.program_id(0)
    nb = pl.num_programs(0)

    def _issue(j):
        @pl.when(jnp.logical_and(j < nb, first_ref[j] == 1))
        def _():
            pltpu.make_async_copy(
                w_hbm.at[be_ref[j]], wbuf.at[slot_ref[j]], sems.at[slot_ref[j]]
            ).start()

    @pl.when(i == 0)
    def _():
        for j in range(_LOOKAHEAD):
            _issue(jnp.int32(j))

    _issue(i + _LOOKAHEAD)

    @pl.when(first_ref[i] == 1)
    def _():
        pltpu.make_async_copy(
            w_hbm.at[be_ref[i]], wbuf.at[slot_ref[i]], sems.at[slot_ref[i]]
        ).wait()

    s = slot_ref[i]
    a = x_ref[...]
    g = jax.lax.dot_general(
        a, wbuf[s, 0], (((1,), (1,)), ((), ())), preferred_element_type=jnp.float32
    )
    u = jax.lax.dot_general(
        a, wbuf[s, 1], (((1,), (1,)), ((), ())), preferred_element_type=jnp.float32
    )
    o_ref[...] = g * jax.nn.sigmoid(g) * u


def kernel(local_hidden_states, up_weight, full_topk_ids):
    T, K = local_hidden_states.shape
    E, N, _ = up_weight.shape
    topk = full_topk_ids.shape[1]
    M = T * topk
    N2 = N // 2

    ids2d = full_topk_ids.reshape(_SUB, _LANE).astype(jnp.int32)

    M_pad = M + E * _BM
    num_blocks = M_pad // _BM

    dest2d, bexp, first, slot = pl.pallas_call(
        functools.partial(_routing_kernel, E=E),
        out_shape=(
            jax.ShapeDtypeStruct((_SUB, _LANE), jnp.int32),
            jax.ShapeDtypeStruct((1, _LANE), jnp.int32),
            jax.ShapeDtypeStruct((1, _LANE), jnp.int32),
            jax.ShapeDtypeStruct((1, _LANE), jnp.int32),
        ),
    )(ids2d)
    dest = dest2d.reshape(M)

    dest_t = dest.reshape(T, topk).T
    info = plsc.get_sparse_core_info()
    nw = info.num_cores * info.num_subcores
    t_per_w = T // nw

    @functools.partial(
        pl.kernel,
        mesh=plsc.VectorSubcoreMesh(core_axis_name="c", subcore_axis_name="s"),
        out_type=jax.ShapeDtypeStruct((M_pad, K), jnp.float32),
        scratch_types=[
            pltpu.VMEM((t_per_w,), jnp.int32),
            pltpu.VMEM((t_per_w,), jnp.int32),
            pltpu.VMEM((t_per_w, K), jnp.float32),
            pltpu.SemaphoreType.DMA,
            pltpu.SemaphoreType.DMA,
        ],
    )
    def _dispatch(x_hbm, dest_hbm, out_hbm, idx0, idx1, xv, sem0, sem1):
        wid = lax.axis_index("s") * info.num_cores + lax.axis_index("c")
        base = wid * t_per_w
        pltpu.sync_copy(dest_hbm.at[0, pl.ds(base, t_per_w)], idx0)
        pltpu.sync_copy(dest_hbm.at[1, pl.ds(base, t_per_w)], idx1)
        pltpu.sync_copy(x_hbm.at[pl.ds(base, t_per_w)], xv)
        c0 = pltpu.async_copy(xv, out_hbm.at[idx0], sem0)
        c1 = pltpu.async_copy(xv, out_hbm.at[idx1], sem1)
        c0.wait()
        c1.wait()

    x_sorted = _dispatch(local_hidden_states, dest_t)

    w4 = up_weight.reshape(E, 2, N2, K)

    grid_spec = pltpu.PrefetchScalarGridSpec(
        num_scalar_prefetch=3,
        grid=(num_blocks,),
        in_specs=[
            pl.BlockSpec((_BM, K), lambda i, be, fi, sl: (i, 0)),
            pl.BlockSpec(memory_space=pl.ANY),
        ],
        out_specs=pl.BlockSpec((_BM, N2), lambda i, be, fi, sl: (i, 0)),
        scratch_shapes=[
            pltpu.VMEM((_NBUF, 2, N2, K), jnp.float32),
            pltpu.SemaphoreType.DMA((_NBUF,)),
        ],
    )
    out_sorted = pl.pallas_call(
        _gemm_silu_kernel,
        grid_spec=grid_spec,
        out_shape=jax.ShapeDtypeStruct((M_pad, N2), jnp.float32),
        compiler_params=pltpu.CompilerParams(
            dimension_semantics=("arbitrary",),
        ),
    )(bexp[0], first[0], slot[0], x_sorted, w4)

    return out_sorted[dest]
